# baseline (speedup 1.0000x reference)
import functools

import jax
import jax.numpy as jnp
from jax import lax
from jax.experimental import pallas as pl
from jax.experimental.pallas import tpu as pltpu
from jax.experimental.pallas import tpu_sc as plsc

B = 32
N = 32768
D = 8192
K = 16
NC = 2
NS = 16
GCH = 1024
NCHUNK = D // GCH


def _tree_reduce(op, xs):
    xs = list(xs)
    while len(xs) > 1:
        nxt = [op(xs[i], xs[i + 1]) for i in range(0, len(xs) - 1, 2)]
        if len(xs) % 2:
            nxt.append(xs[-1])
        xs = nxt
    return xs[0]


def _body(x_hbm, det_hbm, out_hbm, xrow, stat, dbufs, sems, xsem):
    wid = lax.axis_index("s") * NC + lax.axis_index("c")

    xcopy = pltpu.async_copy(x_hbm.at[wid], xrow, xsem)
    copies = [None, None]
    copies[0] = pltpu.async_copy(
        det_hbm.at[:, pl.ds(0, GCH)], dbufs[0], sems[0])

    zeros = jnp.zeros((K,), jnp.int32)

    @plsc.parallel_loop(0, N // K, unroll=16)
    def _zero(i):
        stat[pl.ds(i * K, K)] = zeros

    xcopy.wait()
    ones = jnp.ones((K,), jnp.int32)

    for cidx in range(NCHUNK):
        cur = cidx % 2
        copies[cur].wait()
        if cidx + 1 < NCHUNK:
            copies[1 - cur] = pltpu.async_copy(
                det_hbm.at[:, pl.ds((cidx + 1) * GCH, GCH)],
                dbufs[1 - cur], sems[1 - cur])
        dbuf = dbufs[cur]

        @plsc.parallel_loop(0, GCH // K, step=2)
        def _group(g0):
          for g in (g0, g0 + 1):
            base = g * K
            ids = [dbuf[j, pl.ds(base, K)] for j in range(K)]
            vals = [plsc.load_gather(xrow, [ids[j]]) for j in range(K)]
            pairs = [(vals[j], j) for j in range(K)]
            while len(pairs) > 1:
                nxt = []
                for i in range(0, len(pairs), 2):
                    (av, aj), (bv, bj) = pairs[i], pairs[i + 1]
                    take_a = av >= bv
                    nxt.append((jnp.where(take_a, av, bv),
                                jnp.where(take_a, aj, bj)))
                pairs = nxt
            wmin = pairs[0][1]
            for j in range(K):
                plsc.addupdate_scatter(stat, [ids[j]], ones, mask=wmin != j)

    @plsc.parallel_loop(0, N // K, unroll=8)
    def _fin(i):
        s = stat[pl.ds(i * K, K)]
        xrow[pl.ds(i * K, K)] = jnp.where(s == 0, 1.0, 0.0)

    pltpu.sync_copy(xrow, out_hbm.at[wid])


@jax.jit
def kernel(x, detectors):
    run = pl.kernel(
        _body,
        out_type=jax.ShapeDtypeStruct((B, N), jnp.float32),
        mesh=plsc.VectorSubcoreMesh(
            core_axis_name="c", subcore_axis_name="s",
            num_cores=NC, num_subcores=NS,
        ),
        compiler_params=pltpu.CompilerParams(
            needs_layout_passes=False, disable_bounds_checks=True),
        scratch_types=[
            pltpu.VMEM((N,), jnp.float32),
            pltpu.VMEM((N,), jnp.int32),
            [pltpu.VMEM((K, GCH), jnp.int32) for _ in range(2)],
            [pltpu.SemaphoreType.DMA for _ in range(2)],
            pltpu.SemaphoreType.DMA,
        ],
    )
    det_t = detectors.T
    return run(x, det_t)


# speedup vs baseline: 1.1094x; 1.1094x over previous
import jax
import jax.numpy as jnp
from jax import lax
from jax.experimental import pallas as pl
from jax.experimental.pallas import tpu as pltpu
from jax.experimental.pallas import tpu_sc as plsc

B = 32
N = 32768
D = 8192
K = 16
NC = 2
NS = 16
GCH = 1024
NCHUNK = D // GCH


def _body(x_hbm, det_hbm, out_hbm, xrow, stat, dbufs, sems, xsem):
    wid = lax.axis_index("s") * NC + lax.axis_index("c")

    xcopy = pltpu.async_copy(x_hbm.at[wid], xrow, xsem)
    copies = [None, None]
    copies[0] = pltpu.async_copy(
        det_hbm.at[:, pl.ds(0, GCH)], dbufs[0], sems[0])

    zeros = jnp.zeros((K,), jnp.int32)

    @plsc.parallel_loop(0, N // K, unroll=16)
    def _zero(i):
        stat[pl.ds(i * K, K)] = zeros

    xcopy.wait()
    ones = jnp.ones((K,), jnp.int32)

    for cidx in range(NCHUNK):
        cur = cidx % 2
        copies[cur].wait()
        if cidx + 1 < NCHUNK:
            copies[1 - cur] = pltpu.async_copy(
                det_hbm.at[:, pl.ds((cidx + 1) * GCH, GCH)],
                dbufs[1 - cur], sems[1 - cur])
        dbuf = dbufs[cur]

        @plsc.parallel_loop(0, GCH // K)
        def _group(g):
            base = g * K
            ids = [dbuf[j, pl.ds(base, K)] for j in range(K)]
            vals = [plsc.load_gather(xrow, [ids[j]]) for j in range(K)]
            pairs = [(vals[j], j) for j in range(K)]
            while len(pairs) > 1:
                nxt = []
                for i in range(0, len(pairs), 2):
                    (av, aj), (bv, bj) = pairs[i], pairs[i + 1]
                    take_a = av >= bv
                    nxt.append((jnp.where(take_a, av, bv),
                                jnp.where(take_a, aj, bj)))
                pairs = nxt
            wmin = pairs[0][1]
            for j in range(K):
                plsc.addupdate_scatter(stat, [ids[j]], ones, mask=wmin != j)

    @plsc.parallel_loop(0, N // K, unroll=8)
    def _fin(i):
        s = stat[pl.ds(i * K, K)]
        xrow[pl.ds(i * K, K)] = jnp.where(s == 0, 1.0, 0.0)

    pltpu.sync_copy(xrow, out_hbm.at[wid])


@jax.jit
def kernel(x, detectors):
    run = pl.kernel(
        _body,
        out_type=jax.ShapeDtypeStruct((B, N), jnp.float32),
        mesh=plsc.VectorSubcoreMesh(
            core_axis_name="c", subcore_axis_name="s",
            num_cores=NC, num_subcores=NS,
        ),
        compiler_params=pltpu.CompilerParams(
            needs_layout_passes=False, disable_bounds_checks=True),
        scratch_types=[
            pltpu.VMEM((N,), jnp.float32),
            pltpu.VMEM((N,), jnp.int32),
            [pltpu.VMEM((K, GCH), jnp.int32) for _ in range(2)],
            [pltpu.SemaphoreType.DMA for _ in range(2)],
            pltpu.SemaphoreType.DMA,
        ],
    )
    det_t = detectors.T
    return run(x, det_t)


# speedup vs baseline: 1.1214x; 1.0108x over previous
import jax
import jax.numpy as jnp
from jax import lax
from jax.experimental import pallas as pl
from jax.experimental.pallas import tpu as pltpu
from jax.experimental.pallas import tpu_sc as plsc

B = 32
N = 32768
D = 8192
K = 16
NC = 2
NS = 16
GCH = 1024
NCHUNK = D // GCH


def _body(x_hbm, det_hbm, out_hbm, xrow, stat, dbufs, sems, xsem):
    wid = lax.axis_index("s") * NC + lax.axis_index("c")

    xcopy = pltpu.async_copy(x_hbm.at[wid], xrow, xsem)
    copies = [None, None]
    copies[0] = pltpu.async_copy(
        det_hbm.at[:, pl.ds(0, GCH)], dbufs[0], sems[0])

    zeros = jnp.zeros((K,), jnp.int32)

    @plsc.parallel_loop(0, N // K, unroll=16)
    def _zero(i):
        stat[pl.ds(i * K, K)] = zeros

    xcopy.wait()
    ones = jnp.ones((K,), jnp.int32)

    for cidx in range(NCHUNK):
        cur = cidx % 2
        copies[cur].wait()
        if cidx + 1 < NCHUNK:
            copies[1 - cur] = pltpu.async_copy(
                det_hbm.at[:, pl.ds((cidx + 1) * GCH, GCH)],
                dbufs[1 - cur], sems[1 - cur])
        dbuf = dbufs[cur]

        @plsc.parallel_loop(0, GCH // K)
        def _group(g):
            base = g * K
            ids = [dbuf[j, pl.ds(base, K)] for j in range(K)]
            vals = [plsc.load_gather(xrow, [ids[j]]) for j in range(K)]
            pairs = [(vals[j], j) for j in range(K)]
            while len(pairs) > 1:
                nxt = []
                for i in range(0, len(pairs), 2):
                    (av, aj), (bv, bj) = pairs[i], pairs[i + 1]
                    take_a = av >= bv
                    nxt.append((jnp.where(take_a, av, bv),
                                jnp.where(take_a, aj, bj)))
                pairs = nxt
            wmin = pairs[0][1]
            for j in range(K):
                plsc.addupdate_scatter(stat, [ids[j]], ones, mask=wmin != j)

    seg = N // 8
    ocopies = []
    for p in range(8):
        @plsc.parallel_loop(p * seg // K, (p + 1) * seg // K, unroll=8)
        def _fin(i):
            s = stat[pl.ds(i * K, K)]
            xrow[pl.ds(i * K, K)] = jnp.where(s == 0, 1.0, 0.0)

        ocopies.append(pltpu.async_copy(
            xrow.at[pl.ds(p * seg, seg)],
            out_hbm.at[wid, pl.ds(p * seg, seg)], xsem))
    for c in ocopies:
        c.wait()


@jax.jit
def kernel(x, detectors):
    run = pl.kernel(
        _body,
        out_type=jax.ShapeDtypeStruct((B, N), jnp.float32),
        mesh=plsc.VectorSubcoreMesh(
            core_axis_name="c", subcore_axis_name="s",
            num_cores=NC, num_subcores=NS,
        ),
        compiler_params=pltpu.CompilerParams(
            needs_layout_passes=False, disable_bounds_checks=True),
        scratch_types=[
            pltpu.VMEM((N,), jnp.float32),
            pltpu.VMEM((N,), jnp.int32),
            [pltpu.VMEM((K, GCH), jnp.int32) for _ in range(2)],
            [pltpu.SemaphoreType.DMA for _ in range(2)],
            pltpu.SemaphoreType.DMA,
        ],
    )
    det_t = detectors.T
    return run(x, det_t)
